# baseline (speedup 1.0000x reference)
import functools

import jax
import jax.numpy as jnp
from jax.experimental import pallas as pl

_SQRT1_2 = 0.7071067811865476
_BF16 = jnp.bfloat16


def _gelu_exact(x):
    return 0.5 * x * (1.0 + jax.lax.erf(x * _SQRT1_2))


def _layernorm(x, g, b, eps=1e-5):
    m = jnp.mean(x, axis=-1, keepdims=True)
    v = jnp.mean((x - m) ** 2, axis=-1, keepdims=True)
    return (x - m) / jnp.sqrt(v + eps) * g + b


def _dot(a, b, dims):
    return jax.lax.dot_general(a, b, (dims, ((), ())),
                               preferred_element_type=jnp.float32)


def _stage1_body(rp, z_ref, reg_ref, M_ref, bua_ref, gua_ref,
                 beua_ref, temp_ref, bias_ref, acc_ref):
    i = pl.program_id(0)

    @pl.when(i == 0)
    def _init():
        acc_ref[...] = jnp.zeros_like(acc_ref)

    z = z_ref[...]
    zb = z.astype(_BF16)
    reg = reg_ref[0, 0, :]
    Bsz = z.shape[0]

    zh = _dot(zb, M_ref[...], ((1,), (0,)))
    sim = zh[:, :rp]
    inv_t = 1.0 / temp_ref[0, 0]
    s = sim * inv_t + bias_ref[0, :]
    m = jnp.max(s, axis=1, keepdims=True)
    p = jnp.exp(s - m)
    psum = jnp.sum(p, axis=1)
    col = jax.lax.broadcasted_iota(jnp.int32, (Bsz, rp), 1)
    sel = reg[:, None] == col
    onehot = sel.astype(_BF16)
    w = jnp.sum(jnp.where(sel, p, 0.0), axis=1) / psum
    ew = jnp.exp(w)

    h = _gelu_exact(zh[:, rp:] + bua_ref[0, :])
    h = _layernorm(h, gua_ref[0, :], beua_ref[0, :])

    contrib = (h * ew[:, None]).astype(_BF16)
    extra = jnp.where(col == 0, ew[:, None],
                      jnp.where(col == 1, 1.0, 0.0)).astype(_BF16)
    contrib_ext = jnp.concatenate([contrib, extra], axis=1)
    acc_ref[...] += _dot(onehot, contrib_ext, ((0,), (0,)))


def _stage2_body(h, acc_ref, zf_ref, Wup_ref, bup_ref,
                 gup_ref, beup_ref, Wd1_ref, bd1_ref, gd1_ref, bed1_ref,
                 Wd2_ref, bd2_ref, rraw_ref, uz_ref, db_ref, zfres_ref):
    acc_ext = acc_ref[...]
    cnt = acc_ext[:, h + 1:h + 2]
    mask = cnt > 0
    ssum = jnp.where(mask, acc_ext[:, h:h + 1], 1.0)
    agg = acc_ext[:, :h] / ssum
    proj = _dot(agg.astype(_BF16), Wup_ref[...], ((1,), (0,)))
    proj = _gelu_exact(proj + bup_ref[0, :])
    proj = _layernorm(proj, gup_ref[0, :], beup_ref[0, :])
    zf = zf_ref[...]
    uz = jnp.where(mask, proj, zf)
    uz_ref[...] = uz.astype(_BF16)

    d1 = _dot(uz.astype(_BF16), Wd1_ref[...], ((1,), (0,)))
    d1 = _gelu_exact(d1 + bd1_ref[0, :])
    d1 = _layernorm(d1, gd1_ref[0, :], bed1_ref[0, :])
    db = _dot(d1.astype(_BF16), Wd2_ref[...], ((1,), (0,))) + bd2_ref[0, :]
    db_ref[...] = db.astype(_BF16)

    rw = jax.nn.sigmoid(rraw_ref[0, 0])
    zfres_ref[...] = rw * uz + (1.0 - rw) * zf


def _stage3_body(z_ref, reg_ref, uz_ref, db_ref, temp_ref, rraw_ref,
                 bias_ref, out_ref):
    z = z_ref[...]
    zb = z.astype(_BF16)
    reg = reg_ref[0, 0, :]
    Bsz = z.shape[0]
    Rp = uz_ref.shape[0]

    sim2 = _dot(zb, uz_ref[...], ((1,), (1,)))
    inv_t = 1.0 / temp_ref[0, 0]
    s = sim2 * inv_t + bias_ref[0, :]
    m = jnp.max(s, axis=1, keepdims=True)
    p = jnp.exp(s - m)
    psum = jnp.sum(p, axis=1)
    col = jax.lax.broadcasted_iota(jnp.int32, (Bsz, Rp), 1)
    sel = reg[:, None] == col
    w_recv = jnp.sum(jnp.where(sel, p, 0.0), axis=1) / psum

    gathered = _dot(sel.astype(_BF16), db_ref[...], ((1,), (0,)))
    rw = jax.nn.sigmoid(rraw_ref[0, 0])
    out_ref[...] = rw * (gathered * w_recv[:, None]) + (1.0 - rw) * z


def kernel(z_local, z_fused, regions, W_ua, b_ua, g_ua, be_ua, W_up, b_up,
           g_up, be_up, W_d1, b_d1, g_d1, be_d1, W_d2, b_d2,
           raw_residual_weight, region_prototypes, temperature):
    n, d = z_local.shape
    r = z_fused.shape[0]
    h = W_ua.shape[1]
    rp = 128
    blk = 5000
    assert n % blk == 0
    nb = n // blk

    f32 = jnp.float32
    proto_pt = jnp.zeros((d, rp), _BF16).at[:, :r].set(
        region_prototypes.T.astype(_BF16))
    M = jnp.concatenate([proto_pt, W_ua.astype(_BF16)], axis=1)
    zf_p = jnp.zeros((rp, d), f32).at[:r].set(z_fused)
    reg3 = regions.reshape(nb, 1, blk)
    temp = temperature.reshape(1, 1).astype(f32)
    rraw = raw_residual_weight.reshape(1, 1).astype(f32)
    bias = jnp.where(jnp.arange(rp) < r, 0.0, -1e30).reshape(1, rp)
    row = lambda v: v.reshape(1, -1)

    full = lambda shape: pl.BlockSpec(shape, lambda *_: (0,) * len(shape))
    zspec = pl.BlockSpec((blk, d), lambda i: (i, 0))
    rspec = pl.BlockSpec((1, 1, blk), lambda i: (i, 0, 0))

    acc_ext = pl.pallas_call(
        functools.partial(_stage1_body, rp),
        grid=(nb,),
        in_specs=[zspec, rspec, full((d, rp + h)), full((1, h)),
                  full((1, h)), full((1, h)), full((1, 1)), full((1, rp))],
        out_specs=full((rp, h + rp)),
        out_shape=jax.ShapeDtypeStruct((rp, h + rp), f32),
    )(z_local, reg3, M, row(b_ua), row(g_ua), row(be_ua), temp, bias)

    uz, db, zfres_p = pl.pallas_call(
        functools.partial(_stage2_body, h),
        in_specs=[full((rp, h + rp)), full((rp, d)),
                  full((h, d)), full((1, d)), full((1, d)), full((1, d)),
                  full((d, h)), full((1, h)), full((1, h)), full((1, h)),
                  full((h, d)), full((1, d)), full((1, 1))],
        out_specs=[full((rp, d)), full((rp, d)), full((rp, d))],
        out_shape=[jax.ShapeDtypeStruct((rp, d), _BF16),
                   jax.ShapeDtypeStruct((rp, d), _BF16),
                   jax.ShapeDtypeStruct((rp, d), f32)],
    )(acc_ext, zf_p, W_up.astype(_BF16), row(b_up), row(g_up),
      row(be_up), W_d1.astype(_BF16), row(b_d1), row(g_d1), row(be_d1),
      W_d2.astype(_BF16), row(b_d2), rraw)

    z_local_res = pl.pallas_call(
        _stage3_body,
        grid=(nb,),
        in_specs=[zspec, rspec, full((rp, d)), full((rp, d)), full((1, 1)),
                  full((1, 1)), full((1, rp))],
        out_specs=zspec,
        out_shape=jax.ShapeDtypeStruct((n, d), f32),
    )(z_local, reg3, uz, db, temp, rraw, bias)

    return (z_local_res, zfres_p[:r])


# speedup vs baseline: 1.0450x; 1.0450x over previous
import functools

import jax
import jax.numpy as jnp
from jax.experimental import pallas as pl

_SQRT1_2 = 0.7071067811865476
_BF16 = jnp.bfloat16


def _gelu_exact(x):
    return 0.5 * x * (1.0 + jax.lax.erf(x * _SQRT1_2))


def _layernorm(x, g, b, eps=1e-5):
    m = jnp.mean(x, axis=-1, keepdims=True)
    v = jnp.mean((x - m) ** 2, axis=-1, keepdims=True)
    return (x - m) / jnp.sqrt(v + eps) * g + b


def _dot(a, b, dims):
    return jax.lax.dot_general(a, b, (dims, ((), ())),
                               preferred_element_type=jnp.float32)


def _stage1_body(rp, z_ref, reg_ref, M_ref, bua_ref, gua_ref,
                 beua_ref, temp_ref, bias_ref, acc_ref, zb_ref):
    i = pl.program_id(0)

    @pl.when(i == 0)
    def _init():
        acc_ref[...] = jnp.zeros_like(acc_ref)

    z = z_ref[...]
    zb = z.astype(_BF16)
    zb_ref[...] = zb
    reg = reg_ref[0, 0, :]
    Bsz = z.shape[0]

    zh = _dot(zb, M_ref[...], ((1,), (0,)))
    sim = zh[:, :rp]
    inv_t = 1.0 / temp_ref[0, 0]
    s = sim * inv_t + bias_ref[0, :]
    m = jnp.max(s, axis=1, keepdims=True)
    p = jnp.exp(s - m)
    psum = jnp.sum(p, axis=1)
    col = jax.lax.broadcasted_iota(jnp.int32, (Bsz, rp), 1)
    sel = reg[:, None] == col
    onehot = sel.astype(_BF16)
    w = jnp.sum(jnp.where(sel, p, 0.0), axis=1) / psum
    ew = jnp.exp(w)

    h = _gelu_exact(zh[:, rp:] + bua_ref[0, :])
    h = _layernorm(h, gua_ref[0, :], beua_ref[0, :])

    contrib = (h * ew[:, None]).astype(_BF16)
    extra = jnp.where(col == 0, ew[:, None],
                      jnp.where(col == 1, 1.0, 0.0)).astype(_BF16)
    contrib_ext = jnp.concatenate([contrib, extra], axis=1)
    acc_ref[...] += _dot(onehot, contrib_ext, ((0,), (0,)))


def _stage2_body(h, acc_ref, zf_ref, Wup_ref, bup_ref,
                 gup_ref, beup_ref, Wd1_ref, bd1_ref, gd1_ref, bed1_ref,
                 Wd2_ref, bd2_ref, rraw_ref, uz_ref, db_ref, zfres_ref):
    acc_ext = acc_ref[...]
    cnt = acc_ext[:, h + 1:h + 2]
    mask = cnt > 0
    ssum = jnp.where(mask, acc_ext[:, h:h + 1], 1.0)
    agg = acc_ext[:, :h] / ssum
    proj = _dot(agg.astype(_BF16), Wup_ref[...], ((1,), (0,)))
    proj = _gelu_exact(proj + bup_ref[0, :])
    proj = _layernorm(proj, gup_ref[0, :], beup_ref[0, :])
    zf = zf_ref[...]
    uz = jnp.where(mask, proj, zf)
    uz_ref[...] = uz.astype(_BF16)

    d1 = _dot(uz.astype(_BF16), Wd1_ref[...], ((1,), (0,)))
    d1 = _gelu_exact(d1 + bd1_ref[0, :])
    d1 = _layernorm(d1, gd1_ref[0, :], bed1_ref[0, :])
    db = _dot(d1.astype(_BF16), Wd2_ref[...], ((1,), (0,))) + bd2_ref[0, :]
    db_ref[...] = db.astype(_BF16)

    rw = jax.nn.sigmoid(rraw_ref[0, 0])
    zfres_ref[...] = rw * uz + (1.0 - rw) * zf


def _stage3_body(z_ref, reg_ref, uz_ref, db_ref, temp_ref, rraw_ref,
                 bias_ref, out_ref):
    zb = z_ref[...]
    z = zb.astype(jnp.float32)
    reg = reg_ref[0, 0, :]
    Bsz = z.shape[0]
    Rp = uz_ref.shape[0]

    sim2 = _dot(zb, uz_ref[...], ((1,), (1,)))
    inv_t = 1.0 / temp_ref[0, 0]
    s = sim2 * inv_t + bias_ref[0, :]
    m = jnp.max(s, axis=1, keepdims=True)
    p = jnp.exp(s - m)
    psum = jnp.sum(p, axis=1)
    col = jax.lax.broadcasted_iota(jnp.int32, (Bsz, Rp), 1)
    sel = reg[:, None] == col
    w_recv = jnp.sum(jnp.where(sel, p, 0.0), axis=1) / psum

    gathered = _dot(sel.astype(_BF16), db_ref[...], ((1,), (0,)))
    rw = jax.nn.sigmoid(rraw_ref[0, 0])
    out_ref[...] = rw * (gathered * w_recv[:, None]) + (1.0 - rw) * z


def kernel(z_local, z_fused, regions, W_ua, b_ua, g_ua, be_ua, W_up, b_up,
           g_up, be_up, W_d1, b_d1, g_d1, be_d1, W_d2, b_d2,
           raw_residual_weight, region_prototypes, temperature):
    n, d = z_local.shape
    r = z_fused.shape[0]
    h = W_ua.shape[1]
    rp = 128
    blk = 5000
    assert n % blk == 0
    nb = n // blk

    f32 = jnp.float32
    proto_pt = jnp.zeros((d, rp), _BF16).at[:, :r].set(
        region_prototypes.T.astype(_BF16))
    M = jnp.concatenate([proto_pt, W_ua.astype(_BF16)], axis=1)
    zf_p = jnp.zeros((rp, d), f32).at[:r].set(z_fused)
    reg3 = regions.reshape(nb, 1, blk)
    temp = temperature.reshape(1, 1).astype(f32)
    rraw = raw_residual_weight.reshape(1, 1).astype(f32)
    bias = jnp.where(jnp.arange(rp) < r, 0.0, -1e30).reshape(1, rp)
    row = lambda v: v.reshape(1, -1)

    full = lambda shape: pl.BlockSpec(shape, lambda *_: (0,) * len(shape))
    zspec = pl.BlockSpec((blk, d), lambda i: (i, 0))
    rspec = pl.BlockSpec((1, 1, blk), lambda i: (i, 0, 0))

    acc_ext, zb16 = pl.pallas_call(
        functools.partial(_stage1_body, rp),
        grid=(nb,),
        in_specs=[zspec, rspec, full((d, rp + h)), full((1, h)),
                  full((1, h)), full((1, h)), full((1, 1)), full((1, rp))],
        out_specs=[full((rp, h + rp)), zspec],
        out_shape=[jax.ShapeDtypeStruct((rp, h + rp), f32),
                   jax.ShapeDtypeStruct((n, d), _BF16)],
    )(z_local, reg3, M, row(b_ua), row(g_ua), row(be_ua), temp, bias)

    uz, db, zfres_p = pl.pallas_call(
        functools.partial(_stage2_body, h),
        in_specs=[full((rp, h + rp)), full((rp, d)),
                  full((h, d)), full((1, d)), full((1, d)), full((1, d)),
                  full((d, h)), full((1, h)), full((1, h)), full((1, h)),
                  full((h, d)), full((1, d)), full((1, 1))],
        out_specs=[full((rp, d)), full((rp, d)), full((rp, d))],
        out_shape=[jax.ShapeDtypeStruct((rp, d), _BF16),
                   jax.ShapeDtypeStruct((rp, d), _BF16),
                   jax.ShapeDtypeStruct((rp, d), f32)],
    )(acc_ext, zf_p, W_up.astype(_BF16), row(b_up), row(g_up),
      row(be_up), W_d1.astype(_BF16), row(b_d1), row(g_d1), row(be_d1),
      W_d2.astype(_BF16), row(b_d2), rraw)

    z_local_res = pl.pallas_call(
        _stage3_body,
        grid=(nb,),
        in_specs=[zspec, rspec, full((rp, d)), full((rp, d)), full((1, 1)),
                  full((1, 1)), full((1, rp))],
        out_specs=zspec,
        out_shape=jax.ShapeDtypeStruct((n, d), f32),
    )(zb16, reg3, uz, db, temp, rraw, bias)

    return (z_local_res, zfres_p[:r])
